# initial kernel scaffold (unmeasured)
import jax
import jax.numpy as jnp
from jax import lax
from jax.experimental import pallas as pl
from jax.experimental.pallas import tpu as pltpu


def kernel(
    x,
):
    def body(*refs):
        pass

    out_shape = jax.ShapeDtypeStruct(..., jnp.float32)
    return pl.pallas_call(body, out_shape=out_shape)(...)


# baseline (device time: 152507 ns/iter reference)
import jax
import jax.numpy as jnp
from jax import lax
from jax.experimental import pallas as pl
from jax.experimental.pallas import tpu as pltpu

C = 8


def kernel(x):
    m, n = x.shape
    M = 2 * m
    half = m // 2
    ch = half // C

    def body(x_hbm, out_ref, xtmp, ltmp_sem, y_send, y_recv, x_send, x_recv):
        my_x = lax.axis_index("x")
        my_y = lax.axis_index("y")

        barrier = pltpu.get_barrier_semaphore()
        pl.semaphore_signal(barrier, inc=1, device_id=(my_x, 1 - my_y),
                            device_id_type=pl.DeviceIdType.MESH)
        pl.semaphore_signal(barrier, inc=1, device_id=(1 - my_x, my_y),
                            device_id_type=pl.DeviceIdType.MESH)
        pl.semaphore_wait(barrier, 2)

        def load(src_row, slot):
            return pltpu.make_async_copy(
                x_hbm.at[pl.ds(src_row, ch), :], xtmp.at[slot],
                ltmp_sem.at[slot])

        def stage_half(src_half_row, dst_global_row, after_chunk=None):
            load(src_half_row, 0).start()
            for c in range(C):
                slot = c % 2
                load(src_half_row + c * ch, slot).wait()
                if c + 1 < C:
                    load(src_half_row + (c + 1) * ch, (c + 1) % 2).start()
                out_ref[pl.ds(dst_global_row + c * ch, ch), :] = (
                    xtmp[slot].astype(out_ref.dtype))
                if after_chunk is not None:
                    after_chunk(c)

        own0 = my_y * m + my_x * half
        fwd0 = (1 - my_y) * m + my_x * half

        def rdma_y(c):
            sl = pl.ds(own0 + c * ch, ch)
            return pltpu.make_async_remote_copy(
                src_ref=out_ref.at[sl, :], dst_ref=out_ref.at[sl, :],
                send_sem=y_send.at[c], recv_sem=y_recv.at[c],
                device_id=(my_x, 1 - my_y),
                device_id_type=pl.DeviceIdType.MESH)

        def rdma_x(c):
            sl = pl.ds(fwd0 + c * ch, ch)
            return pltpu.make_async_remote_copy(
                src_ref=out_ref.at[sl, :], dst_ref=out_ref.at[sl, :],
                send_sem=x_send.at[c], recv_sem=x_recv.at[c],
                device_id=(1 - my_x, my_y),
                device_id_type=pl.DeviceIdType.MESH)

        stage_half(my_x * half, own0, after_chunk=lambda c: rdma_y(c).start())

        stage_half((1 - my_x) * half, my_y * m + (1 - my_x) * half)

        for c in range(C):
            rdma_y(c).wait_recv()
            rdma_x(c).start()

        for c in range(C):
            rdma_x(c).wait_recv()
        for c in range(C):
            rdma_y(c).wait_send()
            rdma_x(c).wait_send()

    return pl.pallas_call(
        body,
        out_shape=jax.ShapeDtypeStruct((M, n), jnp.bfloat16),
        in_specs=[pl.BlockSpec(memory_space=pl.ANY)],
        out_specs=pl.BlockSpec(memory_space=pltpu.VMEM),
        scratch_shapes=[
            pltpu.VMEM((2, ch, n), x.dtype),
            pltpu.SemaphoreType.DMA((2,)),
            pltpu.SemaphoreType.DMA((C,)),
            pltpu.SemaphoreType.DMA((C,)),
            pltpu.SemaphoreType.DMA((C,)),
            pltpu.SemaphoreType.DMA((C,)),
        ],
        compiler_params=pltpu.CompilerParams(
            collective_id=0, vmem_limit_bytes=60 * 1024 * 1024),
    )(x)


# device time: 141227 ns/iter; 1.0799x vs baseline; 1.0799x over previous
import jax
import jax.numpy as jnp
from jax import lax
from jax.experimental import pallas as pl
from jax.experimental.pallas import tpu as pltpu

C = 8


def kernel(x):
    m, n = x.shape
    M = 2 * m
    half = m // 2
    ch = half // C

    def body(x_hbm, out_ref, xtmp, ltmp_sem, y_send, y_recv, x_send, x_recv):
        my_x = lax.axis_index("x")
        my_y = lax.axis_index("y")

        barrier = pltpu.get_barrier_semaphore()
        pl.semaphore_signal(barrier, inc=1, device_id=(my_x, 1 - my_y),
                            device_id_type=pl.DeviceIdType.MESH)
        pl.semaphore_signal(barrier, inc=1, device_id=(1 - my_x, my_y),
                            device_id_type=pl.DeviceIdType.MESH)
        pl.semaphore_wait(barrier, 2)

        def load(src_row, slot):
            return pltpu.make_async_copy(
                x_hbm.at[pl.ds(src_row, ch), :], xtmp.at[slot],
                ltmp_sem.at[slot])

        def stage_half(src_half_row, dst_global_row, after_chunk=None):
            load(src_half_row, 0).start()
            for c in range(C):
                slot = c % 2
                load(src_half_row + c * ch, slot).wait()
                if c + 1 < C:
                    load(src_half_row + (c + 1) * ch, (c + 1) % 2).start()
                out_ref[pl.ds(dst_global_row + c * ch, ch), :] = (
                    xtmp[slot].astype(out_ref.dtype))
                if after_chunk is not None:
                    after_chunk(c)

        own0 = my_y * m + my_x * half
        fwd0 = (1 - my_y) * m + my_x * half

        def rdma_y(c):
            sl = pl.ds(own0 + c * ch, ch)
            return pltpu.make_async_remote_copy(
                src_ref=out_ref.at[sl, :], dst_ref=out_ref.at[sl, :],
                send_sem=y_send.at[c], recv_sem=y_recv.at[c],
                device_id=(my_x, 1 - my_y),
                device_id_type=pl.DeviceIdType.MESH)

        def rdma_x(c):
            sl = pl.ds(fwd0 + c * ch, ch)
            return pltpu.make_async_remote_copy(
                src_ref=out_ref.at[sl, :], dst_ref=out_ref.at[sl, :],
                send_sem=x_send.at[c], recv_sem=x_recv.at[c],
                device_id=(1 - my_x, my_y),
                device_id_type=pl.DeviceIdType.MESH)

        stage_half(my_x * half, own0, after_chunk=lambda c: rdma_y(c).start())

        oth_src = (1 - my_x) * half
        oth0 = my_y * m + (1 - my_x) * half
        load(oth_src, 0).start()
        for c in range(C):
            rdma_y(c).wait_recv()
            rdma_x(c).start()
            load(oth_src + c * ch, c % 2).wait()
            if c + 1 < C:
                load(oth_src + (c + 1) * ch, (c + 1) % 2).start()
            out_ref[pl.ds(oth0 + c * ch, ch), :] = (
                xtmp[c % 2].astype(out_ref.dtype))

        for c in range(C):
            rdma_x(c).wait_recv()
        for c in range(C):
            rdma_y(c).wait_send()
            rdma_x(c).wait_send()

    return pl.pallas_call(
        body,
        out_shape=jax.ShapeDtypeStruct((M, n), jnp.bfloat16),
        in_specs=[pl.BlockSpec(memory_space=pl.ANY)],
        out_specs=pl.BlockSpec(memory_space=pltpu.VMEM),
        scratch_shapes=[
            pltpu.VMEM((2, ch, n), x.dtype),
            pltpu.SemaphoreType.DMA((2,)),
            pltpu.SemaphoreType.DMA((C,)),
            pltpu.SemaphoreType.DMA((C,)),
            pltpu.SemaphoreType.DMA((C,)),
            pltpu.SemaphoreType.DMA((C,)),
        ],
        compiler_params=pltpu.CompilerParams(
            collective_id=0, vmem_limit_bytes=60 * 1024 * 1024),
    )(x)


# device time: 140743 ns/iter; 1.0836x vs baseline; 1.0034x over previous
import jax
import jax.numpy as jnp
from jax import lax
from jax.experimental import pallas as pl
from jax.experimental.pallas import tpu as pltpu

CS = 8
R = 4
CC = CS * R


def kernel(x):
    m, n = x.shape
    M = 2 * m
    half = m // 2
    chs = half // CS
    chc = half // CC

    def body(x_hbm, out_ref, xtmp, ltmp_sem, y_send, y_recv, x_send, x_recv):
        my_x = lax.axis_index("x")
        my_y = lax.axis_index("y")

        barrier = pltpu.get_barrier_semaphore()
        pl.semaphore_signal(barrier, inc=1, device_id=(my_x, 1 - my_y),
                            device_id_type=pl.DeviceIdType.MESH)
        pl.semaphore_signal(barrier, inc=1, device_id=(1 - my_x, my_y),
                            device_id_type=pl.DeviceIdType.MESH)
        pl.semaphore_wait(barrier, 2)

        def load(src_row, slot):
            return pltpu.make_async_copy(
                x_hbm.at[pl.ds(src_row, chs), :], xtmp.at[slot],
                ltmp_sem.at[slot])

        own0 = my_y * m + my_x * half
        fwd0 = (1 - my_y) * m + my_x * half

        def rdma_y(c):
            sl = pl.ds(own0 + c * chc, chc)
            return pltpu.make_async_remote_copy(
                src_ref=out_ref.at[sl, :], dst_ref=out_ref.at[sl, :],
                send_sem=y_send.at[c], recv_sem=y_recv.at[c],
                device_id=(my_x, 1 - my_y),
                device_id_type=pl.DeviceIdType.MESH)

        def rdma_x(c):
            sl = pl.ds(fwd0 + c * chc, chc)
            return pltpu.make_async_remote_copy(
                src_ref=out_ref.at[sl, :], dst_ref=out_ref.at[sl, :],
                send_sem=x_send.at[c], recv_sem=x_recv.at[c],
                device_id=(1 - my_x, my_y),
                device_id_type=pl.DeviceIdType.MESH)

        def stage_block(src_half_row, dst_global_row, b, last):
            load(src_half_row + b * chs, b % 2).wait()
            if not last:
                load(src_half_row + (b + 1) * chs, (b + 1) % 2).start()
            out_ref[pl.ds(dst_global_row + b * chs, chs), :] = (
                xtmp[b % 2].astype(out_ref.dtype))

        my_src = my_x * half
        load(my_src, 0).start()
        for b in range(CS):
            stage_block(my_src, own0, b, last=(b == CS - 1))
            for r in range(R):
                rdma_y(b * R + r).start()

        oth_src = (1 - my_x) * half
        oth0 = my_y * m + (1 - my_x) * half
        load(oth_src, 0).start()
        for c in range(CC):
            rdma_y(c).wait_recv()
            rdma_x(c).start()
            if c % R == 0:
                b = c // R
                stage_block(oth_src, oth0, b, last=(b == CS - 1))

        for c in range(CC):
            rdma_x(c).wait_recv()
        for c in range(CC):
            rdma_y(c).wait_send()
            rdma_x(c).wait_send()

    return pl.pallas_call(
        body,
        out_shape=jax.ShapeDtypeStruct((M, n), jnp.bfloat16),
        in_specs=[pl.BlockSpec(memory_space=pl.ANY)],
        out_specs=pl.BlockSpec(memory_space=pltpu.VMEM),
        scratch_shapes=[
            pltpu.VMEM((2, half // CS, n), x.dtype),
            pltpu.SemaphoreType.DMA((2,)),
            pltpu.SemaphoreType.DMA((CC,)),
            pltpu.SemaphoreType.DMA((CC,)),
            pltpu.SemaphoreType.DMA((CC,)),
            pltpu.SemaphoreType.DMA((CC,)),
        ],
        compiler_params=pltpu.CompilerParams(
            collective_id=0, vmem_limit_bytes=60 * 1024 * 1024),
    )(x)
